# Initial kernel scaffold; baseline (speedup 1.0000x reference)
import jax
import jax.numpy as jnp
from jax.experimental import pallas as pl


def kernel(query, reference_points, input_flatten, input_spatial_shapes, input_level_start_index, Wv, bv, Ws, bs, Wa, ba, Wo, bo):
    raise NotImplementedError("write your pallas kernel here")


# speedup vs baseline: 88.4888x; 88.4888x over previous
import functools

import numpy as np
import jax
import jax.numpy as jnp
from jax import lax
from jax.experimental import pallas as pl
from jax.experimental.pallas import tpu as pltpu
from jax.experimental.pallas import tpu_sc as plsc

D_MODEL = 256
N_LEVELS = 4
N_HEADS = 8
N_POINTS = 4
D_HEAD = 32
B = 4
L = 5440
NQ = B * L
NROWS = B * N_HEADS * L
LEVEL_W = (64, 32, 16, 8)
LEVEL_START = (0, 4096, 5120, 5376)

BLK_A = 320
GRID_A = (B, L // BLK_A)
BLK_C = 544

_lane = np.arange(128)
_llev = (_lane // 4) % 4

_G = (np.arange(128)[:, None] // 16 == np.arange(128)[None, :] // 16)
G_MAT = _G.astype(np.float32)

_P = np.zeros((8, 256), np.float32)
for _ln in range(128):
    _P[2 * _llev[_ln] + 0, _ln] = 1.0
    _P[2 * _llev[_ln] + 1, 128 + _ln] = 1.0
P_MAT = _P


def _proj_body(q_ref, r8_ref, f_ref, wv_ref, bv_ref, wsx_ref, bsx_ref,
               wsy_ref, bsy_ref, wa_ref, ba_ref, g_ref, p_ref,
               valt_ref, idx_ref, w_ref):
    b = pl.program_id(0)
    q = q_ref[0]
    v = jnp.dot(f_ref[0], wv_ref[...], preferred_element_type=jnp.float32)
    v = v + bv_ref[...]
    for h in range(N_HEADS):
        valt_ref[0, h] = v[:, h * D_HEAD:(h + 1) * D_HEAD]

    offx = jnp.dot(q, wsx_ref[...], preferred_element_type=jnp.float32) + bsx_ref[...]
    offy = jnp.dot(q, wsy_ref[...], preferred_element_type=jnp.float32) + bsy_ref[...]
    awl = jnp.dot(q, wa_ref[...], preferred_element_type=jnp.float32) + ba_ref[...]
    m = jnp.max(awl, axis=1, keepdims=True)
    e = jnp.exp(awl - m)
    s = jnp.dot(e, g_ref[...], preferred_element_type=jnp.float32,
                precision=lax.Precision.HIGHEST)
    aw = e / s

    refxy = jnp.dot(r8_ref[0], p_ref[...], preferred_element_type=jnp.float32,
                    precision=lax.Precision.HIGHEST)
    refx = refxy[:, :128]
    refy = refxy[:, 128:]

    lane = lax.broadcasted_iota(jnp.int32, (1, 128), 1)
    lev = (lane // 4) % 4
    wl_i = jnp.where(lev == 0, 64,
                     jnp.where(lev == 1, 32, jnp.where(lev == 2, 16, 8)))
    start = jnp.where(lev == 0, 0,
                      jnp.where(lev == 1, 4096,
                                jnp.where(lev == 2, 5120, 5376)))
    wl_f = wl_i.astype(jnp.float32)
    base = (b * N_HEADS + lane // 16) * L + start
    x = refx * wl_f + offx - 0.5
    y = refy * wl_f + offy - 0.5
    x0 = jnp.floor(x)
    y0 = jnp.floor(y)
    fx = x - x0
    fy = y - y0
    wm1 = wl_f - 1.0
    taps = (
        (x0, y0, (1.0 - fx) * (1.0 - fy)),
        (x0, y0 + 1.0, (1.0 - fx) * fy),
        (x0 + 1.0, y0, fx * (1.0 - fy)),
        (x0 + 1.0, y0 + 1.0, fx * fy),
    )
    for t, (xs, ys, bw) in enumerate(taps):
        valid = (xs >= 0.0) & (xs <= wm1) & (ys >= 0.0) & (ys <= wm1)
        xc = jnp.clip(xs, 0.0, wm1).astype(jnp.int32)
        yc = jnp.clip(ys, 0.0, wm1).astype(jnp.int32)
        idx_ref[0, :, t, :] = base + yc * wl_i + xc
        w_ref[0, :, t * 128:(t + 1) * 128] = bw * aw * valid.astype(jnp.float32)


C_Q = 40


def _combine_body(valt_hbm, idx_hbm, w_hbm, out_hbm, idxv, wv, rows, outv, sem):
    nc = 2
    wid = lax.axis_index("s") * nc + lax.axis_index("c")
    q_per_w = NQ // 32
    n_chunks = q_per_w // C_Q
    base_q = wid * q_per_w

    def chunk_body(ch, _):
        qb = base_q + ch * C_Q
        pltpu.sync_copy(idx_hbm.at[pl.ds(qb, C_Q)], idxv)
        pltpu.sync_copy(w_hbm.at[pl.ds(qb * 512, C_Q * 512)], wv)

        def q_body(qi, _):
            copies = [
                pltpu.async_copy(valt_hbm.at[idxv.at[qi, t]],
                                 rows.at[pl.ds(t * 128, 128)], sem)
                for t in range(4)
            ]
            for c in copies:
                c.wait()
            for h in range(N_HEADS):
                def tap_body(t, acc):
                    a0, a1 = acc
                    for j in range(16):
                        i = t * 128 + h * 16 + j
                        ws = plsc.load_gather(
                            wv, [jnp.full((16,), qi * 512 + i, jnp.int32)])
                        a0 = a0 + ws * rows[i, pl.ds(0, 16)]
                        a1 = a1 + ws * rows[i, pl.ds(16, 16)]
                    return (a0, a1)

                z = jnp.zeros((16,), jnp.float32)
                a0, a1 = lax.fori_loop(0, 4, tap_body, (z, z))
                outv[qi, pl.ds(h * 32, 16)] = a0
                outv[qi, pl.ds(h * 32 + 16, 16)] = a1
            return 0

        lax.fori_loop(0, C_Q, q_body, 0)
        pltpu.sync_copy(outv, out_hbm.at[pl.ds(qb, C_Q)])
        return 0

    lax.fori_loop(0, n_chunks, chunk_body, 0)


def _out_body(x_ref, wo_ref, bo_ref, o_ref):
    o_ref[...] = (jnp.dot(x_ref[...], wo_ref[...],
                          preferred_element_type=jnp.float32) + bo_ref[...])


def kernel(query, reference_points, input_flatten, input_spatial_shapes,
           input_level_start_index, Wv, bv, Ws, bs, Wa, ba, Wo, bo):
    r8 = reference_points.reshape(B, L, 8)
    wsx = Ws[:, 0::2]
    wsy = Ws[:, 1::2]
    bsx = bs[0::2].reshape(1, 128)
    bsy = bs[1::2].reshape(1, 128)
    ba2 = ba.reshape(1, 128)
    bv2 = bv.reshape(1, D_MODEL)
    bo2 = bo.reshape(1, D_MODEL)

    full = lambda shape: pl.BlockSpec(shape, lambda b, i: (0,) * len(shape))
    valt, idx, w = pl.pallas_call(
        _proj_body,
        grid=GRID_A,
        in_specs=[
            pl.BlockSpec((1, BLK_A, D_MODEL), lambda b, i: (b, i, 0)),
            pl.BlockSpec((1, BLK_A, 8), lambda b, i: (b, i, 0)),
            pl.BlockSpec((1, BLK_A, D_MODEL), lambda b, i: (b, i, 0)),
            full((D_MODEL, D_MODEL)),
            full((1, D_MODEL)),
            full((D_MODEL, 128)),
            full((1, 128)),
            full((D_MODEL, 128)),
            full((1, 128)),
            full((D_MODEL, 128)),
            full((1, 128)),
            full((128, 128)),
            full((8, 256)),
        ],
        out_specs=[
            pl.BlockSpec((1, N_HEADS, BLK_A, D_HEAD), lambda b, i: (b, 0, i, 0)),
            pl.BlockSpec((1, BLK_A, 4, 128), lambda b, i: (b, i, 0, 0)),
            pl.BlockSpec((1, BLK_A, 512), lambda b, i: (b, i, 0)),
        ],
        out_shape=[
            jax.ShapeDtypeStruct((B, N_HEADS, L, D_HEAD), jnp.float32),
            jax.ShapeDtypeStruct((B, L, 4, 128), jnp.int32),
            jax.ShapeDtypeStruct((B, L, 512), jnp.float32),
        ],
    )(query, r8, input_flatten, Wv, bv2, wsx, bsx, wsy, bsy, Wa, ba2,
      jnp.asarray(G_MAT), jnp.asarray(P_MAT))

    valt2 = valt.reshape(NROWS, D_HEAD)
    idx2 = idx.reshape(NQ, 4, 128)
    w2 = w.reshape(NQ * 512)

    sc = functools.partial(
        pl.kernel,
        out_type=jax.ShapeDtypeStruct((NQ, D_MODEL), jnp.float32),
        mesh=plsc.VectorSubcoreMesh(core_axis_name="c", subcore_axis_name="s"),
        compiler_params=pltpu.CompilerParams(use_tc_tiling_on_sc=False,
                                             needs_layout_passes=False),
        scratch_types=[
            pltpu.VMEM((C_Q, 4, 128), jnp.int32),
            pltpu.VMEM((C_Q * 512,), jnp.float32),
            pltpu.VMEM((512, D_HEAD), jnp.float32),
            pltpu.VMEM((C_Q, D_MODEL), jnp.float32),
            pltpu.SemaphoreType.DMA,
        ],
    )(_combine_body)
    out1 = sc(valt2, idx2, w2)

    out2 = pl.pallas_call(
        _out_body,
        grid=(NQ // BLK_C,),
        in_specs=[
            pl.BlockSpec((BLK_C, D_MODEL), lambda i: (i, 0)),
            pl.BlockSpec((D_MODEL, D_MODEL), lambda i: (0, 0)),
            pl.BlockSpec((1, D_MODEL), lambda i: (0, 0)),
        ],
        out_specs=pl.BlockSpec((BLK_C, D_MODEL), lambda i: (i, 0)),
        out_shape=jax.ShapeDtypeStruct((NQ, D_MODEL), jnp.float32),
    )(out1, Wo, bo2)
    return out2.reshape(B, L, D_MODEL)


# speedup vs baseline: 104.3552x; 1.1793x over previous
import functools

import numpy as np
import jax
import jax.numpy as jnp
from jax import lax
from jax.experimental import pallas as pl
from jax.experimental.pallas import tpu as pltpu
from jax.experimental.pallas import tpu_sc as plsc

D_MODEL = 256
N_LEVELS = 4
N_HEADS = 8
N_POINTS = 4
D_HEAD = 32
B = 4
L = 5440
NQ = B * L
NROWS = B * N_HEADS * L
LEVEL_W = (64, 32, 16, 8)
LEVEL_START = (0, 4096, 5120, 5376)

BLK_A = 320
GRID_A = (B, L // BLK_A)
BLK_C = 544

_lane = np.arange(128)
_llev = (_lane // 4) % 4

_G = (np.arange(128)[:, None] // 16 == np.arange(128)[None, :] // 16)
G_MAT = _G.astype(np.float32)

_P = np.zeros((8, 256), np.float32)
for _ln in range(128):
    _P[2 * _llev[_ln] + 0, _ln] = 1.0
    _P[2 * _llev[_ln] + 1, 128 + _ln] = 1.0
P_MAT = _P


def _proj_body(q_ref, r8_ref, f_ref, wv_ref, bv_ref, wsx_ref, bsx_ref,
               wsy_ref, bsy_ref, wa_ref, ba_ref, g_ref, p_ref,
               valt_ref, idx_ref, w_ref):
    b = pl.program_id(0)
    q = q_ref[0]
    v = jnp.dot(f_ref[0], wv_ref[...], preferred_element_type=jnp.float32)
    v = v + bv_ref[...]
    for h in range(N_HEADS):
        valt_ref[0, h] = v[:, h * D_HEAD:(h + 1) * D_HEAD]

    offx = jnp.dot(q, wsx_ref[...], preferred_element_type=jnp.float32) + bsx_ref[...]
    offy = jnp.dot(q, wsy_ref[...], preferred_element_type=jnp.float32) + bsy_ref[...]
    awl = jnp.dot(q, wa_ref[...], preferred_element_type=jnp.float32) + ba_ref[...]
    m = jnp.max(awl, axis=1, keepdims=True)
    e = jnp.exp(awl - m)
    s = jnp.dot(e, g_ref[...], preferred_element_type=jnp.float32,
                precision=lax.Precision.HIGHEST)
    aw = e / s

    refxy = jnp.dot(r8_ref[0], p_ref[...], preferred_element_type=jnp.float32,
                    precision=lax.Precision.HIGHEST)
    refx = refxy[:, :128]
    refy = refxy[:, 128:]

    lane = lax.broadcasted_iota(jnp.int32, (1, 128), 1)
    lev = (lane // 4) % 4
    wl_i = jnp.where(lev == 0, 64,
                     jnp.where(lev == 1, 32, jnp.where(lev == 2, 16, 8)))
    start = jnp.where(lev == 0, 0,
                      jnp.where(lev == 1, 4096,
                                jnp.where(lev == 2, 5120, 5376)))
    wl_f = wl_i.astype(jnp.float32)
    base = (b * N_HEADS + lane // 16) * L + start
    x = refx * wl_f + offx - 0.5
    y = refy * wl_f + offy - 0.5
    x0 = jnp.floor(x)
    y0 = jnp.floor(y)
    fx = x - x0
    fy = y - y0
    wm1 = wl_f - 1.0
    taps = (
        (x0, y0, (1.0 - fx) * (1.0 - fy)),
        (x0, y0 + 1.0, (1.0 - fx) * fy),
        (x0 + 1.0, y0, fx * (1.0 - fy)),
        (x0 + 1.0, y0 + 1.0, fx * fy),
    )
    for t, (xs, ys, bw) in enumerate(taps):
        valid = (xs >= 0.0) & (xs <= wm1) & (ys >= 0.0) & (ys <= wm1)
        xc = jnp.clip(xs, 0.0, wm1).astype(jnp.int32)
        yc = jnp.clip(ys, 0.0, wm1).astype(jnp.int32)
        idx_ref[0, :, t, :] = base + yc * wl_i + xc
        w_ref[0, :, t * 128:(t + 1) * 128] = bw * aw * valid.astype(jnp.float32)


C_Q = 40


def _combine_body(valt_hbm, idx_hbm, w_hbm, out_hbm, idxv, wv, rows, outv,
                  sem_a, sem_b):
    nc = 2
    wid = lax.axis_index("s") * nc + lax.axis_index("c")
    q_per_w = NQ // 32
    n_chunks = q_per_w // C_Q
    base_q = wid * q_per_w

    def fire(qi, half, sem):
        for t in range(4):
            pltpu.async_copy(valt_hbm.at[idxv.at[qi, t]],
                             rows.at[pl.ds(half * 512 + t * 128, 128)], sem)

    def drain(half, sem):
        for t in range(4):
            pltpu.make_async_copy(
                valt_hbm.at[pl.ds(0, 128)],
                rows.at[pl.ds(half * 512 + t * 128, 128)], sem).wait()

    def combine(qi, half):
        for h in range(N_HEADS):
            def tap_body(t, acc):
                a0, a1 = acc
                for j in range(16):
                    i = t * 128 + h * 16 + j
                    ws = plsc.load_gather(
                        wv, [jnp.full((16,), qi * 512 + i, jnp.int32)])
                    a0 = a0 + ws * rows[half * 512 + i, pl.ds(0, 16)]
                    a1 = a1 + ws * rows[half * 512 + i, pl.ds(16, 16)]
                return (a0, a1)

            z = jnp.zeros((16,), jnp.float32)
            a0, a1 = lax.fori_loop(0, 4, tap_body, (z, z))
            outv[qi, pl.ds(h * 32, 16)] = a0
            outv[qi, pl.ds(h * 32 + 16, 16)] = a1

    def chunk_body(ch, _):
        qb = base_q + ch * C_Q
        pltpu.sync_copy(idx_hbm.at[pl.ds(qb, C_Q)], idxv)
        pltpu.sync_copy(w_hbm.at[pl.ds(qb * 512, C_Q * 512)], wv)
        fire(0, 0, sem_a)

        def pair_body(k, _):
            qa = 2 * k
            drain(0, sem_a)
            fire(qa + 1, 1, sem_b)
            combine(qa, 0)
            drain(1, sem_b)

            @pl.when(k < C_Q // 2 - 1)
            def _():
                fire(qa + 2, 0, sem_a)

            combine(qa + 1, 1)
            return 0

        lax.fori_loop(0, C_Q // 2, pair_body, 0)
        pltpu.sync_copy(outv, out_hbm.at[pl.ds(qb, C_Q)])
        return 0

    lax.fori_loop(0, n_chunks, chunk_body, 0)


def _out_body(x_ref, wo_ref, bo_ref, o_ref):
    o_ref[...] = (jnp.dot(x_ref[...], wo_ref[...],
                          preferred_element_type=jnp.float32) + bo_ref[...])


def kernel(query, reference_points, input_flatten, input_spatial_shapes,
           input_level_start_index, Wv, bv, Ws, bs, Wa, ba, Wo, bo):
    r8 = reference_points.reshape(B, L, 8)
    wsx = Ws[:, 0::2]
    wsy = Ws[:, 1::2]
    bsx = bs[0::2].reshape(1, 128)
    bsy = bs[1::2].reshape(1, 128)
    ba2 = ba.reshape(1, 128)
    bv2 = bv.reshape(1, D_MODEL)
    bo2 = bo.reshape(1, D_MODEL)

    full = lambda shape: pl.BlockSpec(shape, lambda b, i: (0,) * len(shape))
    valt, idx, w = pl.pallas_call(
        _proj_body,
        grid=GRID_A,
        in_specs=[
            pl.BlockSpec((1, BLK_A, D_MODEL), lambda b, i: (b, i, 0)),
            pl.BlockSpec((1, BLK_A, 8), lambda b, i: (b, i, 0)),
            pl.BlockSpec((1, BLK_A, D_MODEL), lambda b, i: (b, i, 0)),
            full((D_MODEL, D_MODEL)),
            full((1, D_MODEL)),
            full((D_MODEL, 128)),
            full((1, 128)),
            full((D_MODEL, 128)),
            full((1, 128)),
            full((D_MODEL, 128)),
            full((1, 128)),
            full((128, 128)),
            full((8, 256)),
        ],
        out_specs=[
            pl.BlockSpec((1, N_HEADS, BLK_A, D_HEAD), lambda b, i: (b, 0, i, 0)),
            pl.BlockSpec((1, BLK_A, 4, 128), lambda b, i: (b, i, 0, 0)),
            pl.BlockSpec((1, BLK_A, 512), lambda b, i: (b, i, 0)),
        ],
        out_shape=[
            jax.ShapeDtypeStruct((B, N_HEADS, L, D_HEAD), jnp.float32),
            jax.ShapeDtypeStruct((B, L, 4, 128), jnp.int32),
            jax.ShapeDtypeStruct((B, L, 512), jnp.float32),
        ],
    )(query, r8, input_flatten, Wv, bv2, wsx, bsx, wsy, bsy, Wa, ba2,
      jnp.asarray(G_MAT), jnp.asarray(P_MAT))

    valt2 = valt.reshape(NROWS, D_HEAD)
    idx2 = idx.reshape(NQ, 4, 128)
    w2 = w.reshape(NQ * 512)

    sc = functools.partial(
        pl.kernel,
        out_type=jax.ShapeDtypeStruct((NQ, D_MODEL), jnp.float32),
        mesh=plsc.VectorSubcoreMesh(core_axis_name="c", subcore_axis_name="s"),
        compiler_params=pltpu.CompilerParams(use_tc_tiling_on_sc=False,
                                             needs_layout_passes=False),
        scratch_types=[
            pltpu.VMEM((C_Q, 4, 128), jnp.int32),
            pltpu.VMEM((C_Q * 512,), jnp.float32),
            pltpu.VMEM((1024, D_HEAD), jnp.float32),
            pltpu.VMEM((C_Q, D_MODEL), jnp.float32),
            pltpu.SemaphoreType.DMA,
            pltpu.SemaphoreType.DMA,
        ],
    )(_combine_body)
    out1 = sc(valt2, idx2, w2)

    out2 = pl.pallas_call(
        _out_body,
        grid=(NQ // BLK_C,),
        in_specs=[
            pl.BlockSpec((BLK_C, D_MODEL), lambda i: (i, 0)),
            pl.BlockSpec((D_MODEL, D_MODEL), lambda i: (0, 0)),
            pl.BlockSpec((1, D_MODEL), lambda i: (0, 0)),
        ],
        out_specs=pl.BlockSpec((BLK_C, D_MODEL), lambda i: (i, 0)),
        out_shape=jax.ShapeDtypeStruct((NQ, D_MODEL), jnp.float32),
    )(out1, Wo, bo2)
    return out2.reshape(B, L, D_MODEL)


# speedup vs baseline: 111.4538x; 1.0680x over previous
import functools

import numpy as np
import jax
import jax.numpy as jnp
from jax import lax
from jax.experimental import pallas as pl
from jax.experimental.pallas import tpu as pltpu
from jax.experimental.pallas import tpu_sc as plsc

D_MODEL = 256
N_LEVELS = 4
N_HEADS = 8
N_POINTS = 4
D_HEAD = 32
B = 4
L = 5440
NQ = B * L
NROWS = B * N_HEADS * L
LEVEL_W = (64, 32, 16, 8)
LEVEL_START = (0, 4096, 5120, 5376)

BLK_A = 320
GRID_A = (B, L // BLK_A)
BLK_C = 544

_lane = np.arange(128)
_llev = (_lane // 4) % 4

_G = (np.arange(128)[:, None] // 16 == np.arange(128)[None, :] // 16)
G_MAT = _G.astype(np.float32)

_P = np.zeros((8, 256), np.float32)
for _ln in range(128):
    _P[2 * _llev[_ln] + 0, _ln] = 1.0
    _P[2 * _llev[_ln] + 1, 128 + _ln] = 1.0
P_MAT = _P


def _proj_body(q_ref, r8_ref, f_ref, wv_ref, bv_ref, wsx_ref, bsx_ref,
               wsy_ref, bsy_ref, wa_ref, ba_ref, g_ref, p_ref,
               valt_ref, idx_ref, w_ref):
    b = pl.program_id(0)
    q = q_ref[0]
    v = jnp.dot(f_ref[0], wv_ref[...], preferred_element_type=jnp.float32)
    v = v + bv_ref[...]
    for h in range(N_HEADS):
        valt_ref[0, h] = v[:, h * D_HEAD:(h + 1) * D_HEAD]

    offx = jnp.dot(q, wsx_ref[...], preferred_element_type=jnp.float32) + bsx_ref[...]
    offy = jnp.dot(q, wsy_ref[...], preferred_element_type=jnp.float32) + bsy_ref[...]
    awl = jnp.dot(q, wa_ref[...], preferred_element_type=jnp.float32) + ba_ref[...]
    m = jnp.max(awl, axis=1, keepdims=True)
    e = jnp.exp(awl - m)
    s = jnp.dot(e, g_ref[...], preferred_element_type=jnp.float32,
                precision=lax.Precision.HIGHEST)
    aw = e / s

    refxy = jnp.dot(r8_ref[0], p_ref[...], preferred_element_type=jnp.float32,
                    precision=lax.Precision.HIGHEST)
    refx = refxy[:, :128]
    refy = refxy[:, 128:]

    lane = lax.broadcasted_iota(jnp.int32, (1, 128), 1)
    lev = (lane // 4) % 4
    wl_i = jnp.where(lev == 0, 64,
                     jnp.where(lev == 1, 32, jnp.where(lev == 2, 16, 8)))
    start = jnp.where(lev == 0, 0,
                      jnp.where(lev == 1, 4096,
                                jnp.where(lev == 2, 5120, 5376)))
    wl_f = wl_i.astype(jnp.float32)
    base = (b * N_HEADS + lane // 16) * L + start
    x = refx * wl_f + offx - 0.5
    y = refy * wl_f + offy - 0.5
    x0 = jnp.floor(x)
    y0 = jnp.floor(y)
    fx = x - x0
    fy = y - y0
    wm1 = wl_f - 1.0
    taps = (
        (x0, y0, (1.0 - fx) * (1.0 - fy)),
        (x0, y0 + 1.0, (1.0 - fx) * fy),
        (x0 + 1.0, y0, fx * (1.0 - fy)),
        (x0 + 1.0, y0 + 1.0, fx * fy),
    )
    for t, (xs, ys, bw) in enumerate(taps):
        valid = (xs >= 0.0) & (xs <= wm1) & (ys >= 0.0) & (ys <= wm1)
        xc = jnp.clip(xs, 0.0, wm1).astype(jnp.int32)
        yc = jnp.clip(ys, 0.0, wm1).astype(jnp.int32)
        idx_ref[0, :, t, :] = base + yc * wl_i + xc
        w_ref[0, :, t * 128:(t + 1) * 128] = bw * aw * valid.astype(jnp.float32)


C_Q = 40


def _combine_body(valt_hbm, idx_hbm, w_hbm, out_hbm, idxv, wv, rows, outv,
                  sem_a, sem_b):
    nc = 2
    wid = lax.axis_index("s") * nc + lax.axis_index("c")
    q_per_w = NQ // 32
    n_chunks = q_per_w // C_Q
    base_q = wid * q_per_w

    def fire(qi, half, sem):
        for t in range(4):
            pltpu.async_copy(valt_hbm.at[idxv.at[qi, t]],
                             rows.at[pl.ds(half * 512 + t * 128, 128)], sem)

    def drain(half, sem):
        for t in range(4):
            pltpu.make_async_copy(
                valt_hbm.at[pl.ds(0, 128)],
                rows.at[pl.ds(half * 512 + t * 128, 128)], sem).wait()

    def combine(qi, half):
        for h in range(N_HEADS):
            def tap_body(t, acc):
                a0, a1 = acc
                w16 = wv[pl.ds(qi * 512 + t * 128 + h * 16, 16)]
                for j in range(16):
                    i = t * 128 + h * 16 + j
                    ws = w16[j]
                    a0 = a0 + ws * rows[half * 512 + i, pl.ds(0, 16)]
                    a1 = a1 + ws * rows[half * 512 + i, pl.ds(16, 16)]
                return (a0, a1)

            z = jnp.zeros((16,), jnp.float32)
            a0, a1 = lax.fori_loop(0, 4, tap_body, (z, z))
            outv[qi, pl.ds(h * 32, 16)] = a0
            outv[qi, pl.ds(h * 32 + 16, 16)] = a1

    def chunk_body(ch, _):
        qb = base_q + ch * C_Q
        pltpu.sync_copy(idx_hbm.at[pl.ds(qb, C_Q)], idxv)
        pltpu.sync_copy(w_hbm.at[pl.ds(qb * 512, C_Q * 512)], wv)
        fire(0, 0, sem_a)

        def pair_body(k, _):
            qa = 2 * k
            drain(0, sem_a)
            fire(qa + 1, 1, sem_b)
            combine(qa, 0)
            drain(1, sem_b)

            @pl.when(k < C_Q // 2 - 1)
            def _():
                fire(qa + 2, 0, sem_a)

            combine(qa + 1, 1)
            return 0

        lax.fori_loop(0, C_Q // 2, pair_body, 0)
        pltpu.sync_copy(outv, out_hbm.at[pl.ds(qb, C_Q)])
        return 0

    lax.fori_loop(0, n_chunks, chunk_body, 0)


def _out_body(x_ref, wo_ref, bo_ref, o_ref):
    o_ref[...] = (jnp.dot(x_ref[...], wo_ref[...],
                          preferred_element_type=jnp.float32) + bo_ref[...])


def kernel(query, reference_points, input_flatten, input_spatial_shapes,
           input_level_start_index, Wv, bv, Ws, bs, Wa, ba, Wo, bo):
    r8 = reference_points.reshape(B, L, 8)
    wsx = Ws[:, 0::2]
    wsy = Ws[:, 1::2]
    bsx = bs[0::2].reshape(1, 128)
    bsy = bs[1::2].reshape(1, 128)
    ba2 = ba.reshape(1, 128)
    bv2 = bv.reshape(1, D_MODEL)
    bo2 = bo.reshape(1, D_MODEL)

    full = lambda shape: pl.BlockSpec(shape, lambda b, i: (0,) * len(shape))
    valt, idx, w = pl.pallas_call(
        _proj_body,
        grid=GRID_A,
        in_specs=[
            pl.BlockSpec((1, BLK_A, D_MODEL), lambda b, i: (b, i, 0)),
            pl.BlockSpec((1, BLK_A, 8), lambda b, i: (b, i, 0)),
            pl.BlockSpec((1, BLK_A, D_MODEL), lambda b, i: (b, i, 0)),
            full((D_MODEL, D_MODEL)),
            full((1, D_MODEL)),
            full((D_MODEL, 128)),
            full((1, 128)),
            full((D_MODEL, 128)),
            full((1, 128)),
            full((D_MODEL, 128)),
            full((1, 128)),
            full((128, 128)),
            full((8, 256)),
        ],
        out_specs=[
            pl.BlockSpec((1, N_HEADS, BLK_A, D_HEAD), lambda b, i: (b, 0, i, 0)),
            pl.BlockSpec((1, BLK_A, 4, 128), lambda b, i: (b, i, 0, 0)),
            pl.BlockSpec((1, BLK_A, 512), lambda b, i: (b, i, 0)),
        ],
        out_shape=[
            jax.ShapeDtypeStruct((B, N_HEADS, L, D_HEAD), jnp.float32),
            jax.ShapeDtypeStruct((B, L, 4, 128), jnp.int32),
            jax.ShapeDtypeStruct((B, L, 512), jnp.float32),
        ],
    )(query, r8, input_flatten, Wv, bv2, wsx, bsx, wsy, bsy, Wa, ba2,
      jnp.asarray(G_MAT), jnp.asarray(P_MAT))

    valt2 = valt.reshape(NROWS, D_HEAD)
    idx2 = idx.reshape(NQ, 4, 128)
    w2 = w.reshape(NQ * 512)

    sc = functools.partial(
        pl.kernel,
        out_type=jax.ShapeDtypeStruct((NQ, D_MODEL), jnp.float32),
        mesh=plsc.VectorSubcoreMesh(core_axis_name="c", subcore_axis_name="s"),
        compiler_params=pltpu.CompilerParams(use_tc_tiling_on_sc=False,
                                             needs_layout_passes=False),
        scratch_types=[
            pltpu.VMEM((C_Q, 4, 128), jnp.int32),
            pltpu.VMEM((C_Q * 512,), jnp.float32),
            pltpu.VMEM((1024, D_HEAD), jnp.float32),
            pltpu.VMEM((C_Q, D_MODEL), jnp.float32),
            pltpu.SemaphoreType.DMA,
            pltpu.SemaphoreType.DMA,
        ],
    )(_combine_body)
    out1 = sc(valt2, idx2, w2)

    out2 = pl.pallas_call(
        _out_body,
        grid=(NQ // BLK_C,),
        in_specs=[
            pl.BlockSpec((BLK_C, D_MODEL), lambda i: (i, 0)),
            pl.BlockSpec((D_MODEL, D_MODEL), lambda i: (0, 0)),
            pl.BlockSpec((1, D_MODEL), lambda i: (0, 0)),
        ],
        out_specs=pl.BlockSpec((BLK_C, D_MODEL), lambda i: (i, 0)),
        out_shape=jax.ShapeDtypeStruct((NQ, D_MODEL), jnp.float32),
    )(out1, Wo, bo2)
    return out2.reshape(B, L, D_MODEL)
